# baseline (speedup 1.0000x reference)
import functools

import jax
import jax.numpy as jnp
from jax import lax
from jax.experimental import pallas as pl
from jax.experimental.pallas import tpu as pltpu
from jax.experimental.pallas import tpu_sc as plsc

B = 1024
CTX = 20
EMB = 64
LANES = 16


def _make_pool_kernel():
    info = plsc.get_sparse_core_info()
    nc, ns = info.num_cores, info.num_subcores
    nw = nc * ns
    b_per_w = B // nw
    n_idx = b_per_w * CTX
    mesh = plsc.VectorSubcoreMesh(core_axis_name="c", subcore_axis_name="s")

    @functools.partial(
        pl.kernel,
        mesh=mesh,
        out_type=jax.ShapeDtypeStruct((B, EMB), jnp.float32),
        scratch_types=[
            pltpu.VMEM((n_idx,), jnp.int32),
            pltpu.VMEM((n_idx, EMB), jnp.float32),
            pltpu.VMEM((b_per_w, EMB), jnp.float32),
            pltpu.SemaphoreType.DMA,
        ],
        compiler_params=pltpu.CompilerParams(use_tc_tiling_on_sc=False),
    )
    def pool(idx_hbm, table_hbm, out_hbm, idx_v, rows_v, acc_v, sem):
        wid = lax.axis_index("s") * nc + lax.axis_index("c")
        base = wid * n_idx
        pltpu.sync_copy(idx_hbm.at[pl.ds(base, n_idx)], idx_v)
        pltpu.async_copy(table_hbm.at[idx_v], rows_v, sem).wait()

        def row_body(r, carry):
            for c in range(EMB // LANES):
                acc = rows_v[r * CTX, pl.ds(c * LANES, LANES)]
                for j in range(1, CTX):
                    acc = acc + rows_v[r * CTX + j, pl.ds(c * LANES, LANES)]
                acc_v[r, pl.ds(c * LANES, LANES)] = acc * (1.0 / CTX)
            return carry

        lax.fori_loop(0, b_per_w, row_body, 0)
        pltpu.sync_copy(acc_v, out_hbm.at[pl.ds(wid * b_per_w, b_per_w)])

    return pool


_BV = 512


def _matmul_bias(pooled, linear_w, linear_b):
    v = linear_w.shape[1]
    nv = pl.cdiv(v, _BV)

    def mm(x_ref, w_ref, b_ref, o_ref):
        o_ref[...] = (
            jnp.dot(x_ref[...], w_ref[...], preferred_element_type=jnp.float32)
            + b_ref[...]
        )

    return pl.pallas_call(
        mm,
        grid=(nv,),
        in_specs=[
            pl.BlockSpec((B, EMB), lambda i: (0, 0)),
            pl.BlockSpec((EMB, _BV), lambda i: (0, i)),
            pl.BlockSpec((1, _BV), lambda i: (0, i)),
        ],
        out_specs=pl.BlockSpec((B, _BV), lambda i: (0, i)),
        out_shape=jax.ShapeDtypeStruct((B, v), jnp.float32),
    )(pooled, linear_w, linear_b.reshape(1, v))


def kernel(context_idxs, embedding_table, linear_w, linear_b):
    idx_flat = context_idxs.reshape(-1).astype(jnp.int32)
    pooled = _make_pool_kernel()(idx_flat, embedding_table)
    return _matmul_bias(pooled, linear_w, linear_b)


# speedup vs baseline: 1.1350x; 1.1350x over previous
import functools

import jax
import jax.numpy as jnp
from jax import lax
from jax.experimental import pallas as pl
from jax.experimental.pallas import tpu as pltpu
from jax.experimental.pallas import tpu_sc as plsc

B = 1024
CTX = 20
EMB = 64
LANES = 16


def _make_pool_kernel():
    info = plsc.get_sparse_core_info()
    nc, ns = info.num_cores, info.num_subcores
    nw = nc * ns
    b_per_w = B // nw
    n_idx = b_per_w * CTX
    mesh = plsc.VectorSubcoreMesh(core_axis_name="c", subcore_axis_name="s")

    @functools.partial(
        pl.kernel,
        mesh=mesh,
        out_type=jax.ShapeDtypeStruct((B, EMB), jnp.float32),
        scratch_types=[
            pltpu.VMEM((n_idx,), jnp.int32),
            pltpu.VMEM((n_idx, EMB), jnp.float32),
            pltpu.VMEM((b_per_w, EMB), jnp.float32),
            pltpu.SemaphoreType.DMA,
        ],
        compiler_params=pltpu.CompilerParams(use_tc_tiling_on_sc=False),
    )
    def pool(idx_hbm, table_hbm, out_hbm, idx_v, rows_v, acc_v, sem):
        wid = lax.axis_index("s") * nc + lax.axis_index("c")
        base = wid * n_idx
        pltpu.sync_copy(idx_hbm.at[pl.ds(base, n_idx)], idx_v)
        pltpu.async_copy(table_hbm.at[idx_v], rows_v, sem).wait()

        def row_body(r, carry):
            for c in range(EMB // LANES):
                acc = rows_v[r * CTX, pl.ds(c * LANES, LANES)]
                for j in range(1, CTX):
                    acc = acc + rows_v[r * CTX + j, pl.ds(c * LANES, LANES)]
                acc_v[r, pl.ds(c * LANES, LANES)] = acc * (1.0 / CTX)
            return carry

        lax.fori_loop(0, b_per_w, row_body, 0)
        pltpu.sync_copy(acc_v, out_hbm.at[pl.ds(wid * b_per_w, b_per_w)])

    return pool


_BV = 2048


def _matmul_bias(pooled, linear_w, linear_b):
    v = linear_w.shape[1]
    nv = pl.cdiv(v, _BV)

    def mm(x_ref, w_ref, b_ref, o_ref):
        o_ref[...] = (
            jnp.dot(x_ref[...], w_ref[...], preferred_element_type=jnp.float32)
            + b_ref[...]
        )

    return pl.pallas_call(
        mm,
        grid=(nv,),
        in_specs=[
            pl.BlockSpec((B, EMB), lambda i: (0, 0)),
            pl.BlockSpec((EMB, _BV), lambda i: (0, i)),
            pl.BlockSpec((1, _BV), lambda i: (0, i)),
        ],
        out_specs=pl.BlockSpec((B, _BV), lambda i: (0, i)),
        out_shape=jax.ShapeDtypeStruct((B, v), jnp.float32),
        compiler_params=pltpu.CompilerParams(
            dimension_semantics=("parallel",),
        ),
    )(pooled, linear_w, linear_b.reshape(1, v))


def kernel(context_idxs, embedding_table, linear_w, linear_b):
    idx_flat = context_idxs.reshape(-1).astype(jnp.int32)
    pooled = _make_pool_kernel()(idx_flat, embedding_table)
    return _matmul_bias(pooled, linear_w, linear_b)


# speedup vs baseline: 2.3652x; 2.0839x over previous
import functools

import jax
import jax.numpy as jnp
from jax import lax
from jax.experimental import pallas as pl
from jax.experimental.pallas import tpu as pltpu
from jax.experimental.pallas import tpu_sc as plsc

B = 1024
CTX = 20
EMB = 64
LANES = 16


def _make_pool_kernel():
    info = plsc.get_sparse_core_info()
    nc, ns = info.num_cores, info.num_subcores
    nw = nc * ns
    b_per_w = B // nw
    n_idx = b_per_w * CTX
    mesh = plsc.VectorSubcoreMesh(core_axis_name="c", subcore_axis_name="s")

    @functools.partial(
        pl.kernel,
        mesh=mesh,
        out_type=jax.ShapeDtypeStruct((B, EMB), jnp.float32),
        scratch_types=[
            pltpu.VMEM((n_idx,), jnp.int32),
            pltpu.VMEM((n_idx, EMB), jnp.float32),
            pltpu.VMEM((b_per_w, EMB), jnp.float32),
            pltpu.SemaphoreType.DMA,
        ],
        compiler_params=pltpu.CompilerParams(use_tc_tiling_on_sc=False),
    )
    def pool(idx_hbm, table_hbm, out_hbm, idx_v, rows_v, acc_v, sem):
        wid = lax.axis_index("s") * nc + lax.axis_index("c")
        base = wid * n_idx
        pltpu.sync_copy(idx_hbm.at[pl.ds(base, n_idx)], idx_v)
        pltpu.async_copy(table_hbm.at[idx_v], rows_v, sem).wait()

        def row_body(r, carry):
            for c in range(EMB // LANES):
                acc = rows_v[r * CTX, pl.ds(c * LANES, LANES)]
                for j in range(1, CTX):
                    acc = acc + rows_v[r * CTX + j, pl.ds(c * LANES, LANES)]
                acc_v[r, pl.ds(c * LANES, LANES)] = acc * (1.0 / CTX)
            return carry

        lax.fori_loop(0, b_per_w, row_body, 0)
        pltpu.sync_copy(acc_v, out_hbm.at[pl.ds(wid * b_per_w, b_per_w)])

    return pool


_BV = 2048


def _matmul_bias_t(pooled, linear_w, linear_b):
    v = linear_w.shape[1]
    nv = pl.cdiv(v, _BV)

    def mm(x_ref, w_ref, b_ref, o_ref):
        wt = jnp.dot(
            w_ref[...].T, x_ref[...].T, preferred_element_type=jnp.float32
        )
        o_ref[...] = wt + b_ref[...]

    return pl.pallas_call(
        mm,
        grid=(nv,),
        in_specs=[
            pl.BlockSpec((B, EMB), lambda i: (0, 0)),
            pl.BlockSpec((EMB, _BV), lambda i: (0, i)),
            pl.BlockSpec((_BV, 1), lambda i: (i, 0)),
        ],
        out_specs=pl.BlockSpec((_BV, B), lambda i: (i, 0)),
        out_shape=jax.ShapeDtypeStruct((v, B), jnp.float32),
        compiler_params=pltpu.CompilerParams(
            dimension_semantics=("parallel",),
        ),
    )(pooled, linear_w, linear_b.reshape(v, 1))


def kernel(context_idxs, embedding_table, linear_w, linear_b):
    idx_flat = context_idxs.reshape(-1).astype(jnp.int32)
    pooled = _make_pool_kernel()(idx_flat, embedding_table)
    return _matmul_bias_t(pooled, linear_w, linear_b).T


# speedup vs baseline: 3.0041x; 1.2701x over previous
import functools

import jax
import jax.numpy as jnp
from jax import lax
from jax.experimental import pallas as pl
from jax.experimental.pallas import tpu as pltpu
from jax.experimental.pallas import tpu_sc as plsc

B = 1024
CTX = 20
EMB = 64
PADDED_EMB = 128
LANES = 16


def _make_pool_kernel():
    info = plsc.get_sparse_core_info()
    nc, ns = info.num_cores, info.num_subcores
    nw = nc * ns
    b_per_w = B // nw
    n_idx = b_per_w * CTX
    mesh = plsc.VectorSubcoreMesh(core_axis_name="c", subcore_axis_name="s")

    @functools.partial(
        pl.kernel,
        mesh=mesh,
        out_type=jax.ShapeDtypeStruct((B, EMB), jnp.float32),
        scratch_types=[
            pltpu.VMEM((n_idx,), jnp.int32),
            pltpu.VMEM((n_idx, PADDED_EMB), jnp.float32),
            pltpu.VMEM((b_per_w, EMB), jnp.float32),
            pltpu.SemaphoreType.DMA,
        ],
        compiler_params=pltpu.CompilerParams(use_tc_tiling_on_sc=False),
    )
    def pool(idx_hbm, table_hbm, out_hbm, idx_v, rows_v, acc_v, sem):
        wid = lax.axis_index("s") * nc + lax.axis_index("c")
        base = wid * n_idx
        pltpu.sync_copy(idx_hbm.at[pl.ds(base, n_idx)], idx_v)
        pltpu.async_copy(table_hbm.at[idx_v], rows_v, sem).wait()

        def row_body(r, carry):
            for c in range(EMB // LANES):
                acc = rows_v[r * CTX, pl.ds(c * LANES, LANES)]
                for j in range(1, CTX):
                    acc = acc + rows_v[r * CTX + j, pl.ds(c * LANES, LANES)]
                acc_v[r, pl.ds(c * LANES, LANES)] = acc * (1.0 / CTX)
            return carry

        lax.fori_loop(0, b_per_w, row_body, 0)
        pltpu.sync_copy(acc_v, out_hbm.at[pl.ds(wid * b_per_w, b_per_w)])

    return pool


_BV = 2048


def _matmul_bias_t(pooled_t, linear_w, linear_b):
    v = linear_w.shape[1]
    nv = pl.cdiv(v, _BV)

    def mm(xt_ref, w_ref, b_ref, o_ref):
        wt_xt = lax.dot_general(
            w_ref[...],
            xt_ref[...],
            (((0,), (0,)), ((), ())),
            preferred_element_type=jnp.float32,
        )
        o_ref[...] = wt_xt + b_ref[...].T

    return pl.pallas_call(
        mm,
        grid=(nv,),
        in_specs=[
            pl.BlockSpec((EMB, B), lambda i: (0, 0)),
            pl.BlockSpec((EMB, _BV), lambda i: (0, i)),
            pl.BlockSpec((1, _BV), lambda i: (0, i)),
        ],
        out_specs=pl.BlockSpec((_BV, B), lambda i: (i, 0)),
        out_shape=jax.ShapeDtypeStruct((v, B), jnp.float32),
        compiler_params=pltpu.CompilerParams(
            dimension_semantics=("parallel",),
        ),
    )(pooled_t, linear_w, linear_b.reshape(1, v))


def kernel(context_idxs, embedding_table, linear_w, linear_b):
    idx_flat = context_idxs.reshape(-1).astype(jnp.int32)
    table128 = jnp.pad(embedding_table, ((0, 0), (0, PADDED_EMB - EMB)))
    pooled = _make_pool_kernel()(idx_flat, table128)
    return _matmul_bias_t(pooled.T, linear_w, linear_b).T


# speedup vs baseline: 3.1713x; 1.0557x over previous
import functools

import jax
import jax.numpy as jnp
from jax import lax
from jax.experimental import pallas as pl
from jax.experimental.pallas import tpu as pltpu
from jax.experimental.pallas import tpu_sc as plsc

B = 1024
CTX = 20
EMB = 64
VOCAB_ = 100000
LANES = 16


def _make_pool_kernel():
    info = plsc.get_sparse_core_info()
    nc, ns = info.num_cores, info.num_subcores
    nw = nc * ns
    dims_per_w = EMB // nw
    mesh = plsc.VectorSubcoreMesh(core_axis_name="c", subcore_axis_name="s")

    @functools.partial(
        pl.kernel,
        mesh=mesh,
        out_type=jax.ShapeDtypeStruct((EMB, B), jnp.float32),
        scratch_types=[
            pltpu.VMEM((CTX, B), jnp.int32),
            pltpu.VMEM((VOCAB_,), jnp.float32),
            pltpu.VMEM((B,), jnp.float32),
        ],
        compiler_params=pltpu.CompilerParams(
            use_tc_tiling_on_sc=False, needs_layout_passes=False
        ),
    )
    def pool(idx_hbm, table_t_hbm, out_hbm, idx_v, row_v, acc_v):
        wid = lax.axis_index("s") * nc + lax.axis_index("c")
        pltpu.sync_copy(idx_hbm, idx_v)
        for p in range(dims_per_w):
            e = wid * dims_per_w + p
            pltpu.sync_copy(table_t_hbm.at[e], row_v)

            def group_body(g, carry):
                base = g * LANES
                acc = jnp.zeros((LANES,), jnp.float32)
                for j in range(CTX):
                    idx16 = idx_v[j, pl.ds(base, LANES)]
                    acc = acc + plsc.load_gather(row_v, [idx16])
                acc_v[pl.ds(base, LANES)] = acc * (1.0 / CTX)
                return carry

            lax.fori_loop(0, B // LANES, group_body, 0)
            pltpu.sync_copy(acc_v, out_hbm.at[e])

    return pool


_BV = 2048


def _matmul_bias_t(pooled_t, linear_w, linear_b):
    v = linear_w.shape[1]
    nv = pl.cdiv(v, _BV)

    def mm(xt_ref, w_ref, b_ref, o_ref):
        wt_xt = lax.dot_general(
            w_ref[...],
            xt_ref[...],
            (((0,), (0,)), ((), ())),
            preferred_element_type=jnp.float32,
        )
        o_ref[...] = wt_xt + b_ref[...].T

    return pl.pallas_call(
        mm,
        grid=(nv,),
        in_specs=[
            pl.BlockSpec((EMB, B), lambda i: (0, 0)),
            pl.BlockSpec((EMB, _BV), lambda i: (0, i)),
            pl.BlockSpec((1, _BV), lambda i: (0, i)),
        ],
        out_specs=pl.BlockSpec((_BV, B), lambda i: (i, 0)),
        out_shape=jax.ShapeDtypeStruct((v, B), jnp.float32),
        compiler_params=pltpu.CompilerParams(
            dimension_semantics=("parallel",),
        ),
    )(pooled_t, linear_w, linear_b.reshape(1, v))


def kernel(context_idxs, embedding_table, linear_w, linear_b):
    idx_t = context_idxs.T.astype(jnp.int32)
    pooled_t = _make_pool_kernel()(idx_t, embedding_table.T)
    return _matmul_bias_t(pooled_t, linear_w, linear_b).T


# speedup vs baseline: 3.7912x; 1.1955x over previous
import functools

import jax
import jax.numpy as jnp
from jax import lax
from jax.experimental import pallas as pl
from jax.experimental.pallas import tpu as pltpu
from jax.experimental.pallas import tpu_sc as plsc

B = 1024
CTX = 20
EMB = 64
VOCAB_ = 100000
LANES = 16


def _make_pool_kernel():
    info = plsc.get_sparse_core_info()
    nc, ns = info.num_cores, info.num_subcores
    nw = nc * ns
    dims_per_w = EMB // nw
    mesh = plsc.VectorSubcoreMesh(core_axis_name="c", subcore_axis_name="s")

    @functools.partial(
        pl.kernel,
        mesh=mesh,
        out_type=jax.ShapeDtypeStruct((EMB, B), jnp.float32),
        scratch_types=[
            pltpu.VMEM((CTX, B), jnp.int32),
            pltpu.VMEM((VOCAB_,), jnp.float32),
            pltpu.VMEM((B,), jnp.float32),
        ],
        compiler_params=pltpu.CompilerParams(
            use_tc_tiling_on_sc=True, needs_layout_passes=False
        ),
    )
    def pool(idx_hbm, table_t_hbm, out_hbm, idx_v, row_v, acc_v):
        wid = lax.axis_index("s") * nc + lax.axis_index("c")
        pltpu.sync_copy(idx_hbm, idx_v)
        for p in range(dims_per_w):
            e = wid * dims_per_w + p
            pltpu.sync_copy(table_t_hbm.at[e], row_v)

            def group_body(g, carry):
                base = g * LANES
                acc = jnp.zeros((LANES,), jnp.float32)
                for j in range(CTX):
                    idx16 = idx_v[j, pl.ds(base, LANES)]
                    acc = acc + plsc.load_gather(row_v, [idx16])
                acc_v[pl.ds(base, LANES)] = acc * (1.0 / CTX)
                return carry

            lax.fori_loop(0, B // LANES, group_body, 0)
            pltpu.sync_copy(acc_v, out_hbm.at[e])

    return pool


_BV = 2048


def _matmul_bias_t(pooled_t, linear_w, linear_b):
    v = linear_w.shape[1]
    nv = pl.cdiv(v, _BV)

    def mm(xt_ref, w_ref, b_ref, o_ref):
        wt_xt = lax.dot_general(
            w_ref[...],
            xt_ref[...],
            (((0,), (0,)), ((), ())),
            preferred_element_type=jnp.float32,
        )
        o_ref[...] = wt_xt + b_ref[...].T

    return pl.pallas_call(
        mm,
        grid=(nv,),
        in_specs=[
            pl.BlockSpec((EMB, B), lambda i: (0, 0)),
            pl.BlockSpec((EMB, _BV), lambda i: (0, i)),
            pl.BlockSpec((1, _BV), lambda i: (0, i)),
        ],
        out_specs=pl.BlockSpec((_BV, B), lambda i: (i, 0)),
        out_shape=jax.ShapeDtypeStruct((v, B), jnp.float32),
        compiler_params=pltpu.CompilerParams(
            dimension_semantics=("parallel",),
        ),
    )(pooled_t, linear_w, linear_b.reshape(1, v))


def kernel(context_idxs, embedding_table, linear_w, linear_b):
    idx_t = context_idxs.T.astype(jnp.int32)
    pooled_t = _make_pool_kernel()(idx_t, embedding_table.T)
    return _matmul_bias_t(pooled_t, linear_w, linear_b).T


# speedup vs baseline: 3.8321x; 1.0108x over previous
import functools

import jax
import jax.numpy as jnp
from jax import lax
from jax.experimental import pallas as pl
from jax.experimental.pallas import tpu as pltpu
from jax.experimental.pallas import tpu_sc as plsc

B = 1024
CTX = 20
EMB = 64
VOCAB_ = 100000
LANES = 16


def _make_pool_kernel():
    info = plsc.get_sparse_core_info()
    nc, ns = info.num_cores, info.num_subcores
    nw = nc * ns
    dims_per_w = EMB // nw
    mesh = plsc.VectorSubcoreMesh(core_axis_name="c", subcore_axis_name="s")

    @functools.partial(
        pl.kernel,
        mesh=mesh,
        out_type=jax.ShapeDtypeStruct((EMB, B), jnp.float32),
        scratch_types=[
            pltpu.VMEM((CTX, B), jnp.int32),
            pltpu.VMEM((VOCAB_,), jnp.float32),
            pltpu.VMEM((B,), jnp.float32),
        ],
        compiler_params=pltpu.CompilerParams(
            use_tc_tiling_on_sc=True, needs_layout_passes=False
        ),
    )
    def pool(idx_hbm, table_t_hbm, out_hbm, idx_v, row_v, acc_v):
        wid = lax.axis_index("s") * nc + lax.axis_index("c")
        pltpu.sync_copy(idx_hbm, idx_v)
        for p in range(dims_per_w):
            e = wid * dims_per_w + p
            pltpu.sync_copy(table_t_hbm.at[e], row_v)

            def group_body(g, carry):
                base = g * LANES
                acc = jnp.zeros((LANES,), jnp.float32)
                for j in range(CTX):
                    idx16 = idx_v[j, pl.ds(base, LANES)]
                    acc = acc + plsc.load_gather(row_v, [idx16])
                acc_v[pl.ds(base, LANES)] = acc * (1.0 / CTX)
                return carry

            lax.fori_loop(0, B // LANES, group_body, 0)
            pltpu.sync_copy(acc_v, out_hbm.at[e])

    return pool


_BV = 4096


def _matmul_bias_t(pooled_t, linear_w, linear_b):
    v = linear_w.shape[1]
    nv = pl.cdiv(v, _BV)

    def mm(xt_ref, w_ref, b_ref, o_ref):
        wt_xt = lax.dot_general(
            w_ref[...],
            xt_ref[...],
            (((0,), (0,)), ((), ())),
            preferred_element_type=jnp.float32,
        )
        o_ref[...] = wt_xt + b_ref[...].T

    return pl.pallas_call(
        mm,
        grid=(nv,),
        in_specs=[
            pl.BlockSpec((EMB, B), lambda i: (0, 0)),
            pl.BlockSpec((EMB, _BV), lambda i: (0, i)),
            pl.BlockSpec((1, _BV), lambda i: (0, i)),
        ],
        out_specs=pl.BlockSpec((_BV, B), lambda i: (i, 0)),
        out_shape=jax.ShapeDtypeStruct((v, B), jnp.float32),
        compiler_params=pltpu.CompilerParams(
            dimension_semantics=("parallel",),
        ),
    )(pooled_t, linear_w, linear_b.reshape(1, v))


def kernel(context_idxs, embedding_table, linear_w, linear_b):
    idx_t = context_idxs.T.astype(jnp.int32)
    pooled_t = _make_pool_kernel()(idx_t, embedding_table.T)
    return _matmul_bias_t(pooled_t, linear_w, linear_b).T
